# Initial kernel scaffold; baseline (speedup 1.0000x reference)
import jax
import jax.numpy as jnp
from jax.experimental import pallas as pl


def kernel(inc_w1, inc_b1, inc_g1, inc_be1, inc_w2, inc_b2, inc_g2, inc_be2, down1_w1, down1_b1, down1_g1, down1_be1, down1_w2, down1_b2, down1_g2, down1_be2, down2_w1, down2_b1, down2_g1, down2_be1, down2_w2, down2_b2, down2_g2, down2_be2, up3_w1, up3_b1, up3_g1, up3_be1, up3_w2, up3_b2, up3_g2, up3_be2, up4_w1, up4_b1, up4_g1, up4_be1, up4_w2, up4_b2, up4_g2, up4_be2, out_w, out_b, x):
    raise NotImplementedError("write your pallas kernel here")


# speedup vs baseline: 7.0719x; 7.0719x over previous
import functools

import jax
import jax.numpy as jnp
import numpy as np
from jax.experimental import pallas as pl
from jax.experimental.pallas import tpu as pltpu

_BN_EPS = 1e-5
_VMEM = 64 * 1024 * 1024


def _rup(v, m):
    return (v + m - 1) // m * m


def _cparams(sem):
    return pltpu.CompilerParams(dimension_semantics=sem,
                                vmem_limit_bytes=_VMEM)


def _pick_th(h, wp):
    best = 1
    for t in range(1, h + 1):
        if h % t == 0 and t * wp <= 4608:
            best = t
    return best


def _conv_tile(ext, wp, wd, rows, b_ref, w_ref, y_ref, st_ref):
    cout = y_ref.shape[2]
    t = jax.lax.broadcasted_iota(jnp.int32, (rows, 1), 0)
    outmask = (jnp.remainder(t, wp) < wd).astype(jnp.float32)
    acc = jnp.zeros((rows, cout), jnp.float32)
    for dy in range(3):
        for dx in range(3):
            off = dy * wp + dx
            acc = acc + jnp.dot(ext[off:off + rows, :], w_ref[3 * dy + dx],
                                preferred_element_type=jnp.float32)
    acc = (acc + b_ref[...]) * outmask
    y_ref[0] = acc.astype(y_ref.dtype)
    s = jnp.sum(acc, axis=0, keepdims=True)
    ss = jnp.sum(acc * acc, axis=0, keepdims=True)
    st_ref[0, 0] = jnp.concatenate([s, ss], axis=0)


def _conv_plain_kernel(x_ref, w_ref, b_ref, y_ref, st_ref,
                       *, wp, wd, rows, erows):
    base = pl.program_id(1) * rows
    ext = x_ref[0, pl.ds(base, erows), :]
    _conv_tile(ext, wp, wd, rows, b_ref, w_ref, y_ref, st_ref)


def _conv_fused_kernel(x_ref, insc_ref, insh_ref, w_ref, b_ref,
                       y_ref, st_ref, xact_ref, *, wp, wd, hwp, rows, erows):
    r = x_ref.shape[1]

    @pl.when(pl.program_id(1) == 0)
    def _():
        pos = 0
        while pos < r:
            c = min(rows, r - pos)
            ti = (jax.lax.broadcasted_iota(jnp.int32, (c, 1), 0)
                  + (pos - (wp + 1)))
            m = ((ti >= 0) & (ti < hwp)
                 & (jnp.remainder(ti, wp) < wd)).astype(jnp.float32)
            xa = x_ref[0, pos:pos + c, :].astype(jnp.float32)
            xa = jnp.maximum(xa * insc_ref[...] + insh_ref[...], 0.0) * m
            xact_ref[pos:pos + c, :] = xa.astype(jnp.bfloat16)
            pos += c

    base = pl.program_id(1) * rows
    ext = xact_ref[pl.ds(base, erows), :]
    _conv_tile(ext, wp, wd, rows, b_ref, w_ref, y_ref, st_ref)


def _conv(xflat, w, b, insc, insh, *, h, wd, th):
    n, r, cin = xflat.shape
    cout = w.shape[-1]
    wp = wd + 2
    nb = h // th
    rows = th * wp
    hwp = h * wp
    erows = _rup(rows + 2 * wp + 2, 8)
    fused = insc is not None

    wk = w.reshape(9, cin, cout).astype(jnp.bfloat16)
    bk = b.reshape(1, cout).astype(jnp.float32)

    flops = 2 * n * hwp * cout * 9 * cin
    cost = pl.CostEstimate(flops=flops, transcendentals=0,
                           bytes_accessed=2 * (n * r * cin + n * hwp * cout))

    common = dict(
        out_shape=(jax.ShapeDtypeStruct((n, hwp, cout), jnp.bfloat16),
                   jax.ShapeDtypeStruct((n, nb, 2, cout), jnp.float32)),
        grid=(n, nb),
        out_specs=(pl.BlockSpec((1, rows, cout), lambda i, j: (i, j, 0)),
                   pl.BlockSpec((1, 1, 2, cout), lambda i, j: (i, j, 0, 0))),
        compiler_params=_cparams(("parallel", "arbitrary")),
        cost_estimate=cost,
    )

    if fused:
        y, st = pl.pallas_call(
            functools.partial(_conv_fused_kernel, wp=wp, wd=wd, hwp=hwp,
                              rows=rows, erows=erows),
            in_specs=[
                pl.BlockSpec((1, r, cin), lambda i, j: (i, 0, 0)),
                pl.BlockSpec((1, cin), lambda i, j: (0, 0)),
                pl.BlockSpec((1, cin), lambda i, j: (0, 0)),
                pl.BlockSpec((9, cin, cout), lambda i, j: (0, 0, 0)),
                pl.BlockSpec((1, cout), lambda i, j: (0, 0)),
            ],
            scratch_shapes=[pltpu.VMEM((r, cin), jnp.bfloat16)],
            **common,
        )(xflat, insc.reshape(1, cin).astype(jnp.float32),
          insh.reshape(1, cin).astype(jnp.float32), wk, bk)
    else:
        y, st = pl.pallas_call(
            functools.partial(_conv_plain_kernel, wp=wp, wd=wd,
                              rows=rows, erows=erows),
            in_specs=[
                pl.BlockSpec((1, r, cin), lambda i, j: (i, 0, 0)),
                pl.BlockSpec((9, cin, cout), lambda i, j: (0, 0, 0)),
                pl.BlockSpec((1, cout), lambda i, j: (0, 0)),
            ],
            **common,
        )(xflat, wk, bk)
    return y, st


def _bn_params(st, gamma, beta, cnt):
    s = jnp.sum(st[:, :, 0, :], axis=(0, 1))
    ss = jnp.sum(st[:, :, 1, :], axis=(0, 1))
    mean = s / cnt
    var = jnp.maximum(ss / cnt - mean * mean, 0.0)
    sc = gamma * jax.lax.rsqrt(var + _BN_EPS)
    sh = beta - mean * sc
    return sc, sh


def _flat_r(h, wp):
    return _rup((h + 2) * wp + 16, 8)


def _nhwc_to_flat(a, h, wd):
    n, _, _, c = a.shape
    wp = wd + 2
    r = _flat_r(h, wp)
    a = jnp.pad(a, ((0, 0), (1, 1), (1, 1), (0, 0)))
    a = a.reshape(n, (h + 2) * wp, c)
    return jnp.pad(a, ((0, 0), (0, r - (h + 2) * wp), (0, 0)))


def _y_to_flat(y, h, wd):
    n, hwp, c = y.shape
    wp = wd + 2
    r = _flat_r(h, wp)
    return jnp.pad(y, ((0, 0), (wp + 1, r - hwp - wp - 1), (0, 0)))


def _act_kernel(y_ref, sc_ref, sh_ref, o_ref):
    v = y_ref[...].astype(jnp.float32)
    o_ref[...] = jnp.maximum(v * sc_ref[...] + sh_ref[...], 0.0).astype(o_ref.dtype)


def _materialize_act(y, sc, sh, h, wd):
    n, hwp, c = y.shape
    wp = wd + 2
    tm = hwp // 8
    act = pl.pallas_call(
        _act_kernel,
        out_shape=jax.ShapeDtypeStruct((n, hwp, c), jnp.bfloat16),
        grid=(n, 8),
        in_specs=[pl.BlockSpec((1, tm, c), lambda i, j: (i, j, 0)),
                  pl.BlockSpec((1, 1, c), lambda i, j: (0, 0, 0)),
                  pl.BlockSpec((1, 1, c), lambda i, j: (0, 0, 0))],
        out_specs=pl.BlockSpec((1, tm, c), lambda i, j: (i, j, 0)),
        compiler_params=_cparams(("parallel", "parallel")),
    )(y, sc.reshape(1, 1, c).astype(jnp.float32),
      sh.reshape(1, 1, c).astype(jnp.float32))
    return act.reshape(n, h, wp, c)[:, :, :wd, :]


def _maxpool2(a):
    n, h, w, c = a.shape
    return a.reshape(n, h // 2, 2, w // 2, 2, c).max(axis=(2, 4))


def _upsample2(a):
    n, h, w, c = a.shape
    ho, wo = 2 * h, 2 * w
    af = a.astype(jnp.float32)

    def cw(osz, isz):
        src = jnp.arange(osz, dtype=jnp.float32) * (isz - 1) / (osz - 1)
        i0 = jnp.floor(src).astype(jnp.int32)
        i1 = jnp.minimum(i0 + 1, isz - 1)
        return src - i0.astype(jnp.float32), i0, i1

    fy, y0, y1 = cw(ho, h)
    fx, x0, x1 = cw(wo, w)
    rows = (af[:, y0, :, :] * (1.0 - fy)[None, :, None, None]
            + af[:, y1, :, :] * fy[None, :, None, None])
    out = (rows[:, :, x0, :] * (1.0 - fx)[None, None, :, None]
           + rows[:, :, x1, :] * fx[None, None, :, None])
    return out.astype(jnp.bfloat16)


def _head_kernel(y_ref, r_ref, sc_ref, sh_ref, w_ref, b_ref, o_ref):
    a = jnp.maximum(y_ref[0].astype(jnp.float32) * sc_ref[...] + sh_ref[...],
                    0.0)
    acc = jnp.dot(a.astype(jnp.bfloat16), w_ref[...],
                  preferred_element_type=jnp.float32)
    o_ref[0] = jnp.tanh(acc + b_ref[...]) + r_ref[0]


def _head(y, sc, sh, inp, w, b, h, wd):
    n, hwp, c = y.shape
    wp = wd + 2
    tm = hwp // 8
    r8 = jnp.pad(inp, ((0, 0), (0, 0), (0, 2), (0, 5)))
    r8 = r8.reshape(n, hwp, 8)
    w8 = jnp.zeros((c, 8), jnp.bfloat16).at[:, :3].set(w.astype(jnp.bfloat16))
    b8 = jnp.zeros((1, 8), jnp.float32).at[:, :3].set(
        b.reshape(1, 3).astype(jnp.float32))
    o = pl.pallas_call(
        _head_kernel,
        out_shape=jax.ShapeDtypeStruct((n, hwp, 8), jnp.float32),
        grid=(n, hwp // tm),
        in_specs=[pl.BlockSpec((1, tm, c), lambda i, j: (i, j, 0)),
                  pl.BlockSpec((1, tm, 8), lambda i, j: (i, j, 0)),
                  pl.BlockSpec((1, c), lambda i, j: (0, 0)),
                  pl.BlockSpec((1, c), lambda i, j: (0, 0)),
                  pl.BlockSpec((c, 8), lambda i, j: (0, 0)),
                  pl.BlockSpec((1, 8), lambda i, j: (0, 0))],
        out_specs=pl.BlockSpec((1, tm, 8), lambda i, j: (i, j, 0)),
        compiler_params=_cparams(("parallel", "parallel")),
        cost_estimate=pl.CostEstimate(
            flops=2 * n * hwp * c * 8, transcendentals=n * hwp * 8,
            bytes_accessed=n * hwp * (c * 2 + 64)),
    )(y, r8, sc.reshape(1, c).astype(jnp.float32),
      sh.reshape(1, c).astype(jnp.float32), w8, b8)
    return o.reshape(n, h, wp, 8)[:, :, :wd, :3]


def _double_conv(xflat, w1, b1, g1, be1, w2, b2, g2, be2, *, h, wd, cnt):
    th = _pick_th(h, wd + 2)
    y, st = _conv(xflat, w1, b1, None, None, h=h, wd=wd, th=th)
    sc, sh = _bn_params(st, g1, be1, cnt)
    y, st = _conv(_y_to_flat(y, h, wd), w2, b2, sc, sh, h=h, wd=wd, th=th)
    sc, sh = _bn_params(st, g2, be2, cnt)
    return y, sc, sh


def kernel(inc_w1, inc_b1, inc_g1, inc_be1, inc_w2, inc_b2, inc_g2, inc_be2,
           down1_w1, down1_b1, down1_g1, down1_be1,
           down1_w2, down1_b2, down1_g2, down1_be2,
           down2_w1, down2_b1, down2_g1, down2_be1,
           down2_w2, down2_b2, down2_g2, down2_be2,
           up3_w1, up3_b1, up3_g1, up3_be1, up3_w2, up3_b2, up3_g2, up3_be2,
           up4_w1, up4_b1, up4_g1, up4_be1, up4_w2, up4_b2, up4_g2, up4_be2,
           out_w, out_b, x):
    n, _, h, wd = x.shape
    h2, w2_, h4, w4 = h // 2, wd // 2, h // 4, wd // 4
    inp = jnp.transpose(x, (0, 2, 3, 1)).astype(jnp.float32)

    x0 = jnp.pad(inp.astype(jnp.bfloat16), ((0, 0), (0, 0), (0, 0), (0, 5)))
    w1p = jnp.pad(inc_w1, ((0, 0), (0, 0), (0, 5), (0, 0)))

    y, sc, sh = _double_conv(_nhwc_to_flat(x0, h, wd),
                             w1p, inc_b1, inc_g1, inc_be1,
                             inc_w2, inc_b2, inc_g2, inc_be2,
                             h=h, wd=wd, cnt=n * h * wd)
    x1 = _materialize_act(y, sc, sh, h, wd)

    y, sc, sh = _double_conv(_nhwc_to_flat(_maxpool2(x1), h2, w2_),
                             down1_w1, down1_b1, down1_g1, down1_be1,
                             down1_w2, down1_b2, down1_g2, down1_be2,
                             h=h2, wd=w2_, cnt=n * h2 * w2_)
    x2 = _materialize_act(y, sc, sh, h2, w2_)

    y, sc, sh = _double_conv(_nhwc_to_flat(_maxpool2(x2), h4, w4),
                             down2_w1, down2_b1, down2_g1, down2_be1,
                             down2_w2, down2_b2, down2_g2, down2_be2,
                             h=h4, wd=w4, cnt=n * h4 * w4)
    x3 = _materialize_act(y, sc, sh, h4, w4)

    cat = jnp.concatenate([x2, _upsample2(x3)], axis=-1)
    y, sc, sh = _double_conv(_nhwc_to_flat(cat, h2, w2_),
                             up3_w1, up3_b1, up3_g1, up3_be1,
                             up3_w2, up3_b2, up3_g2, up3_be2,
                             h=h2, wd=w2_, cnt=n * h2 * w2_)
    u3 = _materialize_act(y, sc, sh, h2, w2_)

    cat = jnp.concatenate([x1, _upsample2(u3)], axis=-1)
    y, sc, sh = _double_conv(_nhwc_to_flat(cat, h, wd),
                             up4_w1, up4_b1, up4_g1, up4_be1,
                             up4_w2, up4_b2, up4_g2, up4_be2,
                             h=h, wd=wd, cnt=n * h * wd)

    out = _head(y, sc, sh, inp, out_w, out_b, h, wd)
    return jnp.transpose(out, (0, 3, 1, 2))
